# baseline (speedup 1.0000x reference)
import functools

import jax
import jax.numpy as jnp
from jax import lax
from jax.experimental import pallas as pl
from jax.experimental.pallas import tpu as pltpu
from jax.experimental.pallas import tpu_sc as plsc

N_NODES = 10000
N_EDGES = 320000
D = 128

NC = 2
NS = 16
NW = NC * NS
L = 16

CHUNK = 128
CPW = 80
N_CHUNKS = NW * CPW
N_EDGES_PAD = N_CHUNKS * CHUNK
N_ACC = 10240
STRIPE = N_ACC // NS
LAST_STRIPE = N_NODES - (NS - 1) * STRIPE


def _lane_broadcast(v, lane):
    return lax.broadcast_in_dim(v[lane], (L,), ())


def _linear_body(x_ref, wt_ref, b_ref, o_ref):
    o_ref[...] = (
        jnp.dot(x_ref[...], wt_ref[...], preferred_element_type=jnp.float32,
                precision=lax.Precision.HIGHEST)
        + b_ref[...]
    )


def _combine_body(p_ref, o_ref):
    o_ref[...] = p_ref[0] + p_ref[1]


def _sc_body(src_hbm, dst_hbm, w_hbm, support_hbm, out_hbm,
             src_v0, dst_v0, w_v0, rows_v0,
             src_v1, dst_v1, w_v1, rows_v1, acc_sh, sem0, sem1):
    ci = lax.axis_index("c")
    si = lax.axis_index("s")
    wid = si * NC + ci

    bufs = ((src_v0, dst_v0, w_v0, rows_v0, sem0),
            (src_v1, dst_v1, w_v1, rows_v1, sem1))

    zero16 = jnp.zeros((L,), jnp.float32)

    def zrow(r, carry):
        for j in range(D // L):
            rows_v0[r, j * L:(j + 1) * L] = zero16
        return carry

    lax.fori_loop(0, CHUNK, zrow, 0)
    base = si * STRIPE
    for t in range(STRIPE // CHUNK):
        pltpu.sync_copy(rows_v0, acc_sh.at[pl.ds(base + t * CHUNK, CHUNK)])
    plsc.subcore_barrier()

    base_c = wid * CPW

    def load_meta(i, b):
        sv, dv, wv, _, _ = bufs[b]
        e0 = (base_c + i) * CHUNK
        pltpu.sync_copy(src_hbm.at[pl.ds(e0, CHUNK)], sv)
        pltpu.sync_copy(dst_hbm.at[pl.ds(e0, CHUNK)], dv)
        pltpu.sync_copy(w_hbm.at[pl.ds(e0, CHUNK)], wv)

    def fire_gather(b):
        sv, _, _, rv, sm = bufs[b]
        pltpu.async_copy(support_hbm.at[sv], rv, sm)

    def wait_gather(b):
        sv, _, _, rv, sm = bufs[b]
        pltpu.make_async_copy(support_hbm.at[sv], rv, sm).wait()

    load_meta(0, 0)
    fire_gather(0)

    def halfstep(i, b):
        _, dv, wvr, rv, _ = bufs[b]
        wait_gather(b)

        @pl.when(i + 1 < CPW)
        def _():
            load_meta(i + 1, 1 - b)
            fire_gather(1 - b)

        def grp(g, gc):
            wv = wvr[pl.ds(g * L, L)]
            for r in range(L):
                wb = _lane_broadcast(wv, r)
                row = g * L + r
                for j in range(D // L):
                    sl = pl.ds(j * L, L)
                    rv[row, sl] = rv[row, sl] * wb
            return gc

        lax.fori_loop(0, CHUNK // L, grp, 0)
        pltpu.sync_copy(rv, acc_sh.at[dv], add=True)

    def pair(p, carry):
        halfstep(2 * p, 0)
        halfstep(2 * p + 1, 1)
        return carry

    lax.fori_loop(0, CPW // 2, pair, 0)
    plsc.subcore_barrier()

    @pl.when(si < NS - 1)
    def _():
        pltpu.sync_copy(acc_sh.at[pl.ds(base, STRIPE)],
                        out_hbm.at[ci, pl.ds(base, STRIPE)])

    @pl.when(si == NS - 1)
    def _():
        pltpu.sync_copy(acc_sh.at[pl.ds(base, LAST_STRIPE)],
                        out_hbm.at[ci, pl.ds(base, LAST_STRIPE)])


_sc_call = functools.partial(
    pl.kernel,
    out_type=jax.ShapeDtypeStruct((NC, N_NODES, D), jnp.float32),
    mesh=plsc.VectorSubcoreMesh(core_axis_name="c", subcore_axis_name="s"),
    scratch_types=[
        pltpu.VMEM((CHUNK,), jnp.int32),
        pltpu.VMEM((CHUNK,), jnp.int32),
        pltpu.VMEM((CHUNK,), jnp.float32),
        pltpu.VMEM((CHUNK, D), jnp.float32),
        pltpu.VMEM((CHUNK,), jnp.int32),
        pltpu.VMEM((CHUNK,), jnp.int32),
        pltpu.VMEM((CHUNK,), jnp.float32),
        pltpu.VMEM((CHUNK, D), jnp.float32),
        pltpu.VMEM_SHARED((N_ACC, D), jnp.float32),
        pltpu.SemaphoreType.DMA,
        pltpu.SemaphoreType.DMA,
    ],
)(_sc_body)

_ROWS_BLK = 1000


def kernel(x, edge_index, edge_weight, W, b):
    pad = N_EDGES_PAD - N_EDGES
    src = jnp.concatenate([edge_index[0], jnp.zeros((pad,), jnp.int32)])
    dst = jnp.concatenate([edge_index[1], jnp.zeros((pad,), jnp.int32)])
    wpad = jnp.concatenate([edge_weight, jnp.zeros((pad,), jnp.float32)])
    wt = W.T
    b2 = b.reshape(1, D)

    support = pl.pallas_call(
        _linear_body,
        grid=(N_NODES // _ROWS_BLK,),
        in_specs=[
            pl.BlockSpec((_ROWS_BLK, D), lambda i: (i, 0)),
            pl.BlockSpec((D, D), lambda i: (0, 0)),
            pl.BlockSpec((1, D), lambda i: (0, 0)),
        ],
        out_specs=pl.BlockSpec((_ROWS_BLK, D), lambda i: (i, 0)),
        out_shape=jax.ShapeDtypeStruct((N_NODES, D), jnp.float32),
    )(x, wt, b2)

    partials = _sc_call(src, dst, wpad, support)

    out = pl.pallas_call(
        _combine_body,
        grid=(N_NODES // _ROWS_BLK,),
        in_specs=[pl.BlockSpec((NC, _ROWS_BLK, D), lambda i: (0, i, 0))],
        out_specs=pl.BlockSpec((_ROWS_BLK, D), lambda i: (i, 0)),
        out_shape=jax.ShapeDtypeStruct((N_NODES, D), jnp.float32),
    )(partials)

    return out


# speedup vs baseline: 2.2236x; 2.2236x over previous
import functools

import jax
import jax.numpy as jnp
from jax import lax
from jax.experimental import pallas as pl
from jax.experimental.pallas import tpu as pltpu
from jax.experimental.pallas import tpu_sc as plsc

N_NODES = 10000
N_EDGES = 320000
D = 128

NC = 2
NS = 16
NW = NC * NS
L = 16

CHUNK = 128
N_CHUNKS = N_EDGES // CHUNK
CHUNKS_PER_W = N_CHUNKS // NW
CHUNKS_REM = N_CHUNKS % NW
N_ACC = 10240
STRIPE = N_ACC // NS
LAST_STRIPE = N_NODES - (NS - 1) * STRIPE


def _lane_broadcast(v, lane):
    return lax.broadcast_in_dim(v[lane], (L,), ())


def _linear_body(x_ref, wt_ref, b_ref, o_ref):
    o_ref[...] = (
        jnp.dot(x_ref[...], wt_ref[...], preferred_element_type=jnp.float32,
                precision=lax.Precision.HIGHEST)
        + b_ref[...]
    )


def _combine_body(p_ref, o_ref):
    o_ref[...] = p_ref[0] + p_ref[1]


def _sc_body(src_hbm, dst_hbm, w_hbm, support_hbm, out_hbm,
             src_v0, dst_v0, w_v0, rows_v0,
             src_v1, dst_v1, w_v1, rows_v1, acc_sh, sem0, sem1):
    ci = lax.axis_index("c")
    si = lax.axis_index("s")
    wid = si * NC + ci

    bufs = ((src_v0, dst_v0, w_v0, rows_v0, sem0),
            (src_v1, dst_v1, w_v1, rows_v1, sem1))

    zero16 = jnp.zeros((L,), jnp.float32)

    def zrow(r, carry):
        for j in range(D // L):
            rows_v0[r, j * L:(j + 1) * L] = zero16
        return carry

    lax.fori_loop(0, CHUNK, zrow, 0)
    base = si * STRIPE
    for t in range(STRIPE // CHUNK):
        pltpu.sync_copy(rows_v0, acc_sh.at[pl.ds(base + t * CHUNK, CHUNK)])
    plsc.subcore_barrier()

    n_my = CHUNKS_PER_W + jnp.where(wid < CHUNKS_REM, 1, 0)

    def load_meta(i, b):
        sv, dv, wv, _, _ = bufs[b]
        e0 = (wid + i * NW) * CHUNK
        pltpu.sync_copy(src_hbm.at[pl.ds(e0, CHUNK)], sv)
        pltpu.sync_copy(dst_hbm.at[pl.ds(e0, CHUNK)], dv)
        pltpu.sync_copy(w_hbm.at[pl.ds(e0, CHUNK)], wv)

    def fire_gather(b):
        sv, _, _, rv, sm = bufs[b]
        pltpu.async_copy(support_hbm.at[sv], rv, sm)

    def wait_gather(b):
        sv, _, _, rv, sm = bufs[b]
        pltpu.make_async_copy(support_hbm.at[sv], rv, sm).wait()

    load_meta(0, 0)
    fire_gather(0)

    def halfstep(i, b):
        _, dv, wvr, rv, _ = bufs[b]
        wait_gather(b)

        @pl.when(i + 1 < n_my)
        def _():
            load_meta(i + 1, 1 - b)
            fire_gather(1 - b)

        def grp(g, gc):
            wv = wvr[pl.ds(g * L, L)]
            for r in range(L):
                wb = _lane_broadcast(wv, r)
                row = g * L + r
                for j in range(D // L):
                    sl = pl.ds(j * L, L)
                    rv[row, sl] = rv[row, sl] * wb
            return gc

        lax.fori_loop(0, CHUNK // L, grp, 0)
        pltpu.sync_copy(rv, acc_sh.at[dv], add=True)

    def pair(p, carry):
        halfstep(2 * p, 0)
        halfstep(2 * p + 1, 1)
        return carry

    lax.fori_loop(0, n_my // 2, pair, 0)

    @pl.when((n_my & 1) == 1)
    def _():
        halfstep(n_my - 1, 0)

    plsc.subcore_barrier()

    @pl.when(si < NS - 1)
    def _():
        pltpu.sync_copy(acc_sh.at[pl.ds(base, STRIPE)],
                        out_hbm.at[ci, pl.ds(base, STRIPE)])

    @pl.when(si == NS - 1)
    def _():
        pltpu.sync_copy(acc_sh.at[pl.ds(base, LAST_STRIPE)],
                        out_hbm.at[ci, pl.ds(base, LAST_STRIPE)])


_sc_call = functools.partial(
    pl.kernel,
    out_type=jax.ShapeDtypeStruct((NC, N_NODES, D), jnp.float32),
    mesh=plsc.VectorSubcoreMesh(core_axis_name="c", subcore_axis_name="s"),
    scratch_types=[
        pltpu.VMEM((CHUNK,), jnp.int32),
        pltpu.VMEM((CHUNK,), jnp.int32),
        pltpu.VMEM((CHUNK,), jnp.float32),
        pltpu.VMEM((CHUNK, D), jnp.float32),
        pltpu.VMEM((CHUNK,), jnp.int32),
        pltpu.VMEM((CHUNK,), jnp.int32),
        pltpu.VMEM((CHUNK,), jnp.float32),
        pltpu.VMEM((CHUNK, D), jnp.float32),
        pltpu.VMEM_SHARED((N_ACC, D), jnp.float32),
        pltpu.SemaphoreType.DMA,
        pltpu.SemaphoreType.DMA,
    ],
)(_sc_body)

_ROWS_BLK = 1000


def kernel(x, edge_index, edge_weight, W, b):
    src = edge_index[0]
    dst = edge_index[1]
    wt = W.T
    b2 = b.reshape(1, D)

    support = pl.pallas_call(
        _linear_body,
        grid=(N_NODES // _ROWS_BLK,),
        in_specs=[
            pl.BlockSpec((_ROWS_BLK, D), lambda i: (i, 0)),
            pl.BlockSpec((D, D), lambda i: (0, 0)),
            pl.BlockSpec((1, D), lambda i: (0, 0)),
        ],
        out_specs=pl.BlockSpec((_ROWS_BLK, D), lambda i: (i, 0)),
        out_shape=jax.ShapeDtypeStruct((N_NODES, D), jnp.float32),
    )(x, wt, b2)

    partials = _sc_call(src, dst, edge_weight, support)

    out = pl.pallas_call(
        _combine_body,
        grid=(N_NODES // _ROWS_BLK,),
        in_specs=[pl.BlockSpec((NC, _ROWS_BLK, D), lambda i: (0, i, 0))],
        out_specs=pl.BlockSpec((_ROWS_BLK, D), lambda i: (i, 0)),
        out_shape=jax.ShapeDtypeStruct((N_NODES, D), jnp.float32),
    )(partials)

    return out


# speedup vs baseline: 3.0960x; 1.3924x over previous
import functools

import jax
import jax.numpy as jnp
from jax import lax
from jax.experimental import pallas as pl
from jax.experimental.pallas import tpu as pltpu
from jax.experimental.pallas import tpu_sc as plsc

N_NODES = 10000
N_EDGES = 320000
D = 128

NC = 2
NS = 16
NW = NC * NS
L = 16

CHUNK = 128
N_CHUNKS = N_EDGES // CHUNK
CHUNKS_PER_W = N_CHUNKS // NW
CHUNKS_REM = N_CHUNKS % NW
N_ACC = 10240
STRIPE = N_ACC // NS
LAST_STRIPE = N_NODES - (NS - 1) * STRIPE


def _lane_broadcast(v, lane):
    return lax.broadcast_in_dim(v[lane], (L,), ())


def _linear_body(x_ref, wt_ref, b_ref, o_ref):
    o_ref[...] = (
        jnp.dot(x_ref[...], wt_ref[...], preferred_element_type=jnp.float32,
                precision=lax.Precision.HIGHEST)
        + b_ref[...]
    )


def _combine_body(p_ref, o_ref):
    o_ref[...] = p_ref[0] + p_ref[1]


def _sc_body(edge_hbm, w_hbm, support_hbm, out_hbm,
             src_v0, dst_v0, w_v0, rows_v0,
             src_v1, dst_v1, w_v1, rows_v1, acc_sh, sem0, sem1,
             msem0, msem1):
    ci = lax.axis_index("c")
    si = lax.axis_index("s")
    wid = si * NC + ci

    bufs = ((src_v0, dst_v0, w_v0, rows_v0, sem0, msem0),
            (src_v1, dst_v1, w_v1, rows_v1, sem1, msem1))

    zero16 = jnp.zeros((L,), jnp.float32)

    def zrow(r, carry):
        for j in range(D // L):
            rows_v0[r, j * L:(j + 1) * L] = zero16
        return carry

    lax.fori_loop(0, CHUNK, zrow, 0)
    base = si * STRIPE
    for t in range(STRIPE // CHUNK):
        pltpu.sync_copy(rows_v0, acc_sh.at[pl.ds(base + t * CHUNK, CHUNK)])
    plsc.subcore_barrier()

    n_my = CHUNKS_PER_W + jnp.where(wid < CHUNKS_REM, 1, 0)

    def load_meta(i, b):
        sv, dv, wv, _, sm, msm = bufs[b]
        c = wid + i * NW
        pltpu.async_copy(edge_hbm.at[0, c], sv, sm)
        pltpu.async_copy(edge_hbm.at[1, c], dv, msm)
        pltpu.async_copy(w_hbm.at[c], wv, msm)
        pltpu.make_async_copy(edge_hbm.at[0, c], sv, sm).wait()

    def fire_gather(b):
        sv, _, _, rv, sm = bufs[b][:5]
        pltpu.async_copy(support_hbm.at[sv], rv, sm)

    def wait_gather(b):
        sv, _, _, rv, sm = bufs[b][:5]
        pltpu.make_async_copy(support_hbm.at[sv], rv, sm).wait()

    def wait_meta(i, b):
        _, dv, wv, _, _, msm = bufs[b]
        c = wid + i * NW
        pltpu.make_async_copy(edge_hbm.at[1, c], dv, msm).wait()
        pltpu.make_async_copy(w_hbm.at[c], wv, msm).wait()

    load_meta(0, 0)
    fire_gather(0)

    def halfstep(i, b):
        _, dv, wvr, rv, _, _ = bufs[b]
        wait_gather(b)
        wait_meta(i, b)

        @pl.when(i + 1 < n_my)
        def _():
            load_meta(i + 1, 1 - b)
            fire_gather(1 - b)

        def grp(g, gc):
            wv = wvr[pl.ds(g * L, L)]
            for r in range(L):
                wb = _lane_broadcast(wv, r)
                row = g * L + r
                for j in range(D // L):
                    sl = pl.ds(j * L, L)
                    rv[row, sl] = rv[row, sl] * wb
            return gc

        lax.fori_loop(0, CHUNK // L, grp, 0)
        pltpu.sync_copy(rv, acc_sh.at[dv], add=True)

    def pair(p, carry):
        halfstep(2 * p, 0)
        halfstep(2 * p + 1, 1)
        return carry

    lax.fori_loop(0, n_my // 2, pair, 0)

    @pl.when((n_my & 1) == 1)
    def _():
        halfstep(n_my - 1, 0)

    plsc.subcore_barrier()

    @pl.when(si < NS - 1)
    def _():
        pltpu.sync_copy(acc_sh.at[pl.ds(base, STRIPE)],
                        out_hbm.at[ci, pl.ds(base, STRIPE)])

    @pl.when(si == NS - 1)
    def _():
        pltpu.sync_copy(acc_sh.at[pl.ds(base, LAST_STRIPE)],
                        out_hbm.at[ci, pl.ds(base, LAST_STRIPE)])


_sc_call = functools.partial(
    pl.kernel,
    out_type=jax.ShapeDtypeStruct((NC, N_NODES, D), jnp.float32),
    mesh=plsc.VectorSubcoreMesh(core_axis_name="c", subcore_axis_name="s"),
    scratch_types=[
        pltpu.VMEM((CHUNK,), jnp.int32),
        pltpu.VMEM((CHUNK,), jnp.int32),
        pltpu.VMEM((CHUNK,), jnp.float32),
        pltpu.VMEM((CHUNK, D), jnp.float32),
        pltpu.VMEM((CHUNK,), jnp.int32),
        pltpu.VMEM((CHUNK,), jnp.int32),
        pltpu.VMEM((CHUNK,), jnp.float32),
        pltpu.VMEM((CHUNK, D), jnp.float32),
        pltpu.VMEM_SHARED((N_ACC, D), jnp.float32),
        pltpu.SemaphoreType.DMA,
        pltpu.SemaphoreType.DMA,
        pltpu.SemaphoreType.DMA,
        pltpu.SemaphoreType.DMA,
    ],
)(_sc_body)

_ROWS_BLK = 1000


def kernel(x, edge_index, edge_weight, W, b):
    edge3 = edge_index.reshape(2, N_CHUNKS, CHUNK)
    wts2 = edge_weight.reshape(N_CHUNKS, CHUNK)
    wt = W.T
    b2 = b.reshape(1, D)

    support = pl.pallas_call(
        _linear_body,
        grid=(N_NODES // _ROWS_BLK,),
        in_specs=[
            pl.BlockSpec((_ROWS_BLK, D), lambda i: (i, 0)),
            pl.BlockSpec((D, D), lambda i: (0, 0)),
            pl.BlockSpec((1, D), lambda i: (0, 0)),
        ],
        out_specs=pl.BlockSpec((_ROWS_BLK, D), lambda i: (i, 0)),
        out_shape=jax.ShapeDtypeStruct((N_NODES, D), jnp.float32),
    )(x, wt, b2)

    partials = _sc_call(edge3, wts2, support)

    out = pl.pallas_call(
        _combine_body,
        grid=(N_NODES // _ROWS_BLK,),
        in_specs=[pl.BlockSpec((NC, _ROWS_BLK, D), lambda i: (0, i, 0))],
        out_specs=pl.BlockSpec((_ROWS_BLK, D), lambda i: (i, 0)),
        out_shape=jax.ShapeDtypeStruct((N_NODES, D), jnp.float32),
    )(partials)

    return out
